# Initial kernel scaffold; baseline (speedup 1.0000x reference)
import jax
import jax.numpy as jnp
from jax.experimental import pallas as pl


def kernel(image, question, question_len, embeddings, bboxes, emb_lengths, W_ocr, b_ocr, W_q, b_q):
    raise NotImplementedError("write your pallas kernel here")


# speedup vs baseline: 9.1037x; 9.1037x over previous
import functools

import jax
import jax.numpy as jnp
from jax import lax
from jax.experimental import pallas as pl
from jax.experimental.pallas import tpu as pltpu

B, Q, L, D, HW = 4, 1, 16, 300, 224
N = HW * HW
IDX = 32
BG = 16
R = 16
RT = HW // R
PIX = R * HW


def _prologue_body(emb_ref, q_ref, wo_ref, bo_ref, wq_ref, bq_ref,
                   embT_ref, embx_ref, qout_ref, s_ref):
    e2 = emb_ref[...].reshape(B * L, D)
    h = lax.dot_general(e2, wo_ref[...], (((1,), (1,)), ((), ())),
                        preferred_element_type=jnp.float32) + bo_ref[...]
    h = jnp.maximum(h, 0.0)
    hn = jnp.sqrt(jnp.sum(h * h, axis=1, keepdims=True))
    h = h / jnp.maximum(hn, 1e-12)

    q2 = q_ref[...].reshape(B * Q, D)
    qh = lax.dot_general(q2, wq_ref[...], (((1,), (1,)), ((), ())),
                         preferred_element_type=jnp.float32) + bq_ref[...]
    qh = jnp.maximum(qh, 0.0)
    qn = jnp.sqrt(jnp.sum(qh * qh, axis=1, keepdims=True))
    qh = qh / jnp.maximum(qn, 1e-12)
    qout_ref[...] = qh.reshape(B, Q, D)

    ext = jnp.concatenate(
        [h.reshape(B, L, D), jnp.zeros((B, IDX - L, D), jnp.float32)], axis=1)
    embx_ref[...] = ext
    embT_ref[...] = jnp.transpose(ext, (0, 2, 1))
    s_ref[...] = lax.dot_general(qh.reshape(B, Q, D), ext,
                                 (((2,), (2,)), ((0,), (0,))),
                                 preferred_element_type=jnp.float32)


def _grid_body(bbox_ref, len_ref, embT_ref, wg_ref, cnt_ref):
    b = pl.program_id(0)
    rt = pl.program_id(1)
    p = lax.broadcasted_iota(jnp.int32, (1, PIX), 1)
    r = rt * R + p // HW
    c = p % HW
    idx = jnp.full((1, PIX), BG, jnp.int32)
    elen = len_ref[b]
    for i in range(L):
        x = bbox_ref[b, i, 0]
        y = bbox_ref[b, i, 1]
        x2 = bbox_ref[b, i, 2]
        y2 = bbox_ref[b, i, 3]
        cov = (r >= y) & (r < y2) & (c >= x) & (c < x2) & (i < elen)
        idx = jnp.where(cov, i, idx)
    oh = (lax.broadcasted_iota(jnp.int32, (IDX, PIX), 0) == idx
          ).astype(jnp.float32)
    wg_ref[0] = lax.dot_general(embT_ref[0], oh, (((1,), (0,)), ((), ())),
                                preferred_element_type=jnp.float32)

    cnt = lax.dot_general(jnp.ones((1, PIX), jnp.float32), oh,
                          (((1,), (1,)), ((), ())),
                          preferred_element_type=jnp.float32)

    @pl.when(rt == 0)
    def _():
        cnt_ref[...] = jnp.zeros_like(cnt_ref)

    cnt_ref[0] += cnt


def _final_body(cnt_ref, s_ref, embx_ref, out_ref):
    c = cnt_ref[...].reshape(B, IDX)
    sv = s_ref[...].reshape(B, IDX)
    active = c > 0.0
    m = jnp.max(jnp.where(active, sv, -1e30), axis=1, keepdims=True)
    e = jnp.where(active, jnp.exp(sv - m), 0.0)
    w = c * e
    z = jnp.sum(w, axis=1, keepdims=True)
    coef = w / z
    wa = lax.dot_general(coef, embx_ref[...], (((1,), (1,)), ((0,), (0,))),
                         preferred_element_type=jnp.float32)
    nrm = jnp.sqrt(jnp.sum(wa * wa, axis=0, keepdims=True))
    out_ref[...] = wa / jnp.maximum(nrm, 1e-12)


def kernel(image, question, question_len, embeddings, bboxes, emb_lengths,
           W_ocr, b_ocr, W_q, b_q):
    del image, question_len

    embT, embx, qout, s = pl.pallas_call(
        _prologue_body,
        out_shape=(
            jax.ShapeDtypeStruct((B, D, IDX), jnp.float32),
            jax.ShapeDtypeStruct((B, IDX, D), jnp.float32),
            jax.ShapeDtypeStruct((B, Q, D), jnp.float32),
            jax.ShapeDtypeStruct((B, Q, IDX), jnp.float32),
        ),
    )(embeddings, question, W_ocr, b_ocr.reshape(1, D), W_q, b_q.reshape(1, D))

    wordgrid, counts = pl.pallas_call(
        _grid_body,
        grid=(B, RT),
        in_specs=[
            pl.BlockSpec(memory_space=pltpu.SMEM),
            pl.BlockSpec(memory_space=pltpu.SMEM),
            pl.BlockSpec((1, D, IDX), lambda b, rt: (b, 0, 0)),
        ],
        out_specs=[
            pl.BlockSpec((1, D, PIX), lambda b, rt: (b, 0, rt)),
            pl.BlockSpec((1, 1, IDX), lambda b, rt: (b, 0, 0)),
        ],
        out_shape=(
            jax.ShapeDtypeStruct((B, D, N), jnp.float32),
            jax.ShapeDtypeStruct((B, 1, IDX), jnp.float32),
        ),
    )(bboxes, emb_lengths, embT)

    wavg = pl.pallas_call(
        _final_body,
        out_shape=jax.ShapeDtypeStruct((B, D), jnp.float32),
    )(counts, s, embx)

    return (wavg, qout, wordgrid)
